# baseline (speedup 1.0000x reference)
import functools

import jax
import jax.numpy as jnp
from jax import lax
from jax.experimental import pallas as pl
from jax.experimental.pallas import tpu as pltpu
from jax.experimental.pallas import tpu_sc as plsc

_MIN_AGE = 40
_MAX_AGE = 80
_AGE_WINDOW = 5
_NUM_STRATA = 2 * ((_MAX_AGE - _MIN_AGE) // _AGE_WINDOW)
_NUM_AGE_GROUPS = _NUM_STRATA // 2
_NUM_COLS = _NUM_STRATA + 1
_NUM_CONCEPTS = 100000

_B, _L, _D = 1024, 200, 128
_N = _B * _L

_NC, _NS = 2, 16
_NW = _NC * _NS
_PER_W = _N // _NW
_LANES = 16

_sc_mesh = plsc.VectorSubcoreMesh(core_axis_name="c", subcore_axis_name="s")


_Q = _PER_W // 4
_NCH = 4
_WIN_PER_CH = _Q // _LANES


@functools.partial(
    pl.kernel,
    mesh=_sc_mesh,
    out_type=jax.ShapeDtypeStruct((_N,), jnp.float32),
    scratch_types=[
        pltpu.VMEM((_PER_W,), jnp.int32),
        pltpu.VMEM((_PER_W,), jnp.int32),
        pltpu.VMEM((_PER_W,), jnp.int32),
        pltpu.VMEM((_PER_W,), jnp.int32),
        pltpu.VMEM((_PER_W,), jnp.float32),
        pltpu.SemaphoreType.DMA,
        pltpu.SemaphoreType.DMA((_NCH,)),
    ],
)
def _sc_gather(concept_hbm, age_hbm, gender_hbm, sig_hbm, sel_hbm,
               c_v, a_v, g_v, idx_v, s_v, isem, gsem):
    wid = lax.axis_index("s") * _NC + lax.axis_index("c")
    base = wid * _PER_W

    for ref, hbm in ((c_v, concept_hbm), (a_v, age_hbm), (g_v, gender_hbm)):
        pltpu.make_async_copy(
            hbm.at[pl.ds(base, _PER_W)], ref, isem
        ).start()
    for ref, hbm in ((c_v, concept_hbm), (a_v, age_hbm), (g_v, gender_hbm)):
        pltpu.make_async_copy(
            hbm.at[pl.ds(base, _PER_W)], ref, isem
        ).wait()

    def index_chunk(k):
        def body(j, carry):
            t0 = k * _Q + j * _LANES
            a = a_v[pl.ds(t0, _LANES)]
            g = g_v[pl.ds(t0, _LANES)]
            c = c_v[pl.ds(t0, _LANES)]
            strat = lax.div(a - _MIN_AGE, _AGE_WINDOW) + _NUM_AGE_GROUPS * g
            ok = (a >= _MIN_AGE) & (a <= _MAX_AGE)
            col = jnp.where(ok, strat, _NUM_STRATA)
            idx_v[pl.ds(t0, _LANES)] = col * _NUM_CONCEPTS + c
            return carry
        lax.fori_loop(0, _WIN_PER_CH, body, 0)

    def start_gather(k):
        pltpu.make_async_copy(
            sig_hbm.at[idx_v.at[pl.ds(k * _Q, _Q)]],
            s_v.at[pl.ds(k * _Q, _Q)], gsem.at[k]
        ).start()

    def wait_gather(k):
        pltpu.make_async_copy(
            sig_hbm.at[idx_v.at[pl.ds(k * _Q, _Q)]],
            s_v.at[pl.ds(k * _Q, _Q)], gsem.at[k]
        ).wait()

    for k in range(_NCH):
        index_chunk(k)
        start_gather(k)
    for k in range(_NCH):
        wait_gather(k)

    pltpu.sync_copy(s_v, sel_hbm.at[pl.ds(base, _PER_W)])


_C = 4096
_NB = 3
_S = _N // _C


def _tc_pipe(emb_hbm, noi_hbm, sel_hbm, out_hbm, ebuf, nbuf, sbuf, obuf,
             esem, nsem, ssem, osem):
    def start_read(i, b):
        off = i * _C
        pltpu.make_async_copy(emb_hbm.at[pl.ds(off, _C), :], ebuf.at[b], esem.at[b]).start()
        pltpu.make_async_copy(noi_hbm.at[pl.ds(off, _C), :], nbuf.at[b], nsem.at[b]).start()
        pltpu.make_async_copy(sel_hbm.at[pl.ds(off, _C)], sbuf.at[b], ssem.at[b]).start()

    def wait_read(i, b):
        off = i * _C
        pltpu.make_async_copy(emb_hbm.at[pl.ds(off, _C), :], ebuf.at[b], esem.at[b]).wait()
        pltpu.make_async_copy(noi_hbm.at[pl.ds(off, _C), :], nbuf.at[b], nsem.at[b]).wait()
        pltpu.make_async_copy(sel_hbm.at[pl.ds(off, _C)], sbuf.at[b], ssem.at[b]).wait()

    def start_write(i, b):
        off = i * _C
        pltpu.make_async_copy(obuf.at[b], out_hbm.at[pl.ds(off, _C), :], osem.at[b]).start()

    def wait_write(i, b):
        off = i * _C
        pltpu.make_async_copy(obuf.at[b], out_hbm.at[pl.ds(off, _C), :], osem.at[b]).wait()

    for b in range(_NB - 1):
        start_read(b, b)

    def step(i, carry):
        b = lax.rem(i, _NB)

        @pl.when(i + _NB - 1 < _S)
        def _():
            start_read(i + _NB - 1, lax.rem(i + _NB - 1, _NB))

        wait_read(i, b)

        @pl.when(i >= _NB)
        def _():
            wait_write(i - _NB, b)

        obuf[b] = ebuf[b] + nbuf[b] * sbuf[b][:, None]
        start_write(i, b)
        return carry

    lax.fori_loop(0, _S, step, 0)
    for k in range(_NB):
        i = _S - _NB + k
        wait_write(i, i % _NB)


@jax.jit
def kernel(concept, age, gender, embeddings, noise, sigmas):
    sel = _sc_gather(
        concept.reshape(_N), age.reshape(_N), gender.reshape(_N),
        sigmas.T.reshape(_NUM_CONCEPTS * _NUM_COLS),
    )
    emb2 = embeddings.reshape(_N, _D)
    noi2 = noise.reshape(_N, _D)
    out2 = pl.pallas_call(
        _tc_pipe,
        in_specs=[
            pl.BlockSpec(memory_space=pl.ANY),
            pl.BlockSpec(memory_space=pl.ANY),
            pl.BlockSpec(memory_space=pl.ANY),
        ],
        out_specs=pl.BlockSpec(memory_space=pl.ANY),
        out_shape=jax.ShapeDtypeStruct((_N, _D), jnp.float32),
        scratch_shapes=[
            pltpu.VMEM((_NB, _C, _D), jnp.float32),
            pltpu.VMEM((_NB, _C, _D), jnp.float32),
            pltpu.VMEM((_NB, _C), jnp.float32),
            pltpu.VMEM((_NB, _C, _D), jnp.float32),
            pltpu.SemaphoreType.DMA((_NB,)),
            pltpu.SemaphoreType.DMA((_NB,)),
            pltpu.SemaphoreType.DMA((_NB,)),
            pltpu.SemaphoreType.DMA((_NB,)),
        ],
    )(emb2, noi2, sel)
    return out2.reshape(_B, _L, _D)
